# baseline (speedup 1.0000x reference)
import functools

import jax
import jax.numpy as jnp
import numpy as np
from jax import lax
from jax.experimental import pallas as pl
from jax.experimental.pallas import tpu as pltpu
from jax.experimental.pallas import tpu_sc as plsc

_BATCH = 16384
_FEAT = 32
_LANES = 16
_PACK = 128 // _FEAT

_NC = 2
_NS = 16
_NW = _NC * _NS
_BPW = _BATCH // _NW
_CHUNK = 128
_NCHUNK = _BPW // _CHUNK
_FROWS = _BPW // _PACK
_GROUPS = _BPW // _LANES

_ROWID = np.arange(_BPW, dtype=np.int32)
_AUX = np.stack([
    _ROWID,
    _ROWID // _PACK,
    (_ROWID % _PACK) * _FEAT,
])


@functools.cache
def _build():
    mesh = plsc.VectorSubcoreMesh(core_axis_name="c", subcore_axis_name="s")

    @functools.partial(
        pl.kernel,
        mesh=mesh,
        out_type=jax.ShapeDtypeStruct((_NW, _LANES), jnp.float32),
        scratch_types=[
            pltpu.VMEM((_NCHUNK, _CHUNK), jnp.int32),
            pltpu.VMEM((_BPW,), jnp.int32),
            pltpu.VMEM((3, _BPW), jnp.int32),
            pltpu.VMEM((_BPW, 128), jnp.float32),
            pltpu.VMEM((_FROWS, 128), jnp.float32),
            pltpu.VMEM((_LANES,), jnp.float32),
            pltpu.SemaphoreType.DMA,
            pltpu.SemaphoreType.DMA,
        ],
        compiler_params=pltpu.CompilerParams(needs_layout_passes=False),
    )
    def center_loss_partials(feat_hbm, gidx_hbm, moff_hbm, aux_hbm,
                             centers_hbm, out_hbm, gidx_v, moff_v, aux_v,
                             ctr_v, feat_v, acc_v, gsem, fsem):
        wid = lax.axis_index("s") * _NC + lax.axis_index("c")

        pltpu.sync_copy(gidx_hbm.at[wid], gidx_v)
        fcopy = pltpu.async_copy(
            feat_hbm.at[pl.ds(wid * _FROWS, _FROWS)], feat_v, fsem)
        mcopy = pltpu.async_copy(moff_hbm.at[wid], moff_v, fsem)
        acopy = pltpu.async_copy(aux_hbm, aux_v, fsem)
        gcopies = [
            pltpu.async_copy(
                centers_hbm.at[gidx_v.at[j]],
                ctr_v.at[pl.ds(j * _CHUNK, _CHUNK)],
                gsem)
            for j in range(_NCHUNK)
        ]
        fcopy.wait()
        mcopy.wait()
        acopy.wait()
        for c in gcopies:
            c.wait()

        zeros = jnp.zeros((_LANES,), jnp.float32)

        def step(g, acc):
            rows = aux_v[0, pl.ds(g * _LANES, _LANES)]
            frows = aux_v[1, pl.ds(g * _LANES, _LANES)]
            foffs = aux_v[2, pl.ds(g * _LANES, _LANES)]
            moffs = moff_v[pl.ds(g * _LANES, _LANES)]
            for c in range(_FEAT):
                cvec = jnp.full((_LANES,), c, jnp.int32)
                cv = plsc.load_gather(ctr_v, [rows, moffs + cvec])
                fv = plsc.load_gather(feat_v, [frows, foffs + cvec])
                d = fv - cv
                acc = acc + d * d
            return acc

        acc = lax.fori_loop(0, _GROUPS, step, zeros)
        acc_v[...] = acc
        pltpu.sync_copy(acc_v, out_hbm.at[wid])

    return center_loss_partials


def kernel(features, labels, centers):
    labels = labels.astype(jnp.int32)
    gidx = (labels // _PACK).reshape(_NW, _NCHUNK, _CHUNK)
    moff = ((labels % _PACK) * _FEAT).reshape(_NW, _BPW)
    feat2 = features.reshape(_NW * _FROWS, 128)
    centers2 = centers.reshape(centers.shape[0] // _PACK, 128)
    aux = jnp.asarray(_AUX)
    partials = _build()(feat2, gidx, moff, aux, centers2)
    return jnp.sum(partials) * (0.5 / _BATCH)
